# baseline (speedup 1.0000x reference)
import functools

import jax
import jax.numpy as jnp
from jax import lax
from jax.experimental import pallas as pl
from jax.experimental.pallas import tpu as pltpu
from jax.experimental.pallas import tpu_sc as plsc

S1 = 4
S2 = 12
D = 64
NSEL = 4
NC, NS = 2, 16
NW = NC * NS
BLK = 64


def _build_sc_gather(B, ne, nr):
    bpw = B // NW
    mesh = plsc.VectorSubcoreMesh(
        core_axis_name="c", subcore_axis_name="s",
        num_cores=NC, num_subcores=NS)

    @functools.partial(
        pl.kernel,
        mesh=mesh,
        compiler_params=pltpu.CompilerParams(
            needs_layout_passes=False, use_tc_tiling_on_sc=False),
        out_type=[
            jax.ShapeDtypeStruct((B, D), jnp.float32),
            jax.ShapeDtypeStruct((B, D), jnp.float32),
            jax.ShapeDtypeStruct((B, D), jnp.float32),
            jax.ShapeDtypeStruct((NSEL * B, D), jnp.float32),
            jax.ShapeDtypeStruct((NSEL * B, D), jnp.float32),
        ],
        scratch_types=[
            pltpu.VMEM((bpw, 16), jnp.float32),
            pltpu.VMEM((bpw, 16), jnp.int32),
            pltpu.VMEM((bpw, 16), jnp.int32),
            pltpu.VMEM((bpw,), jnp.int32),
            pltpu.VMEM((bpw,), jnp.int32),
            pltpu.VMEM((bpw,), jnp.int32),
            pltpu.VMEM((NSEL * bpw,), jnp.int32),
            pltpu.VMEM((NSEL * bpw,), jnp.int32),
            pltpu.VMEM((bpw, D), jnp.float32),
            pltpu.VMEM((bpw, D), jnp.float32),
            pltpu.VMEM((bpw, D), jnp.float32),
            pltpu.VMEM((NSEL * bpw, D), jnp.float32),
            pltpu.VMEM((NSEL * bpw, D), jnp.float32),
            pltpu.SemaphoreType.DMA,
        ],
    )
    def sc_gather(sidx_h, pidx_h, oidx_h, time_h, nh_h, nt_h, ent_h, rel_h,
                  es_h, ep_h, eo_h, gh_h, gt_h,
                  time_v, nh_v, nt_v, sidx_v, pidx_v, oidx_v, hidx_v, tidx_v,
                  srow_v, prow_v, orow_v, hrow_v, trow_v, sem):
        wid = lax.axis_index("s") * NC + lax.axis_index("c")
        base = wid * bpw

        pltpu.sync_copy(time_h.at[pl.ds(base, bpw)], time_v)
        pltpu.sync_copy(nh_h.at[pl.ds(base, bpw)], nh_v)
        pltpu.sync_copy(nt_h.at[pl.ds(base, bpw)], nt_v)
        pltpu.sync_copy(sidx_h.at[pl.ds(base, bpw)], sidx_v)
        pltpu.sync_copy(pidx_h.at[pl.ds(base, bpw)], pidx_v)
        pltpu.sync_copy(oidx_h.at[pl.ds(base, bpw)], oidx_v)

        for group in range(bpw // 16):
            rows = jnp.arange(16, dtype=jnp.int32) + (group * 16)
            fv = jnp.zeros((16,), jnp.float32)
            for s in range(1, S1):
                col = jnp.full((16,), s, jnp.int32)
                fv = fv + float(s) * plsc.load_gather(time_v, [rows, col])
            fi = fv.astype(jnp.int32)
            for k in range(NSEL):
                cols = NSEL * fi + k
                dst = rows * NSEL + k
                plsc.store_scatter(hidx_v, [dst],
                                   plsc.load_gather(nh_v, [rows, cols]))
                plsc.store_scatter(tidx_v, [dst],
                                   plsc.load_gather(nt_v, [rows, cols]))

        cps = pltpu.async_copy(ent_h.at[sidx_v], srow_v, sem)
        cpp = pltpu.async_copy(rel_h.at[pidx_v], prow_v, sem)
        cpo = pltpu.async_copy(ent_h.at[oidx_v], orow_v, sem)
        cph = pltpu.async_copy(ent_h.at[hidx_v], hrow_v, sem)
        cpt = pltpu.async_copy(ent_h.at[tidx_v], trow_v, sem)
        cps.wait()
        cpp.wait()
        cpo.wait()
        cph.wait()
        cpt.wait()

        pltpu.sync_copy(srow_v, es_h.at[pl.ds(base, bpw)])
        pltpu.sync_copy(prow_v, ep_h.at[pl.ds(base, bpw)])
        pltpu.sync_copy(orow_v, eo_h.at[pl.ds(base, bpw)])
        pltpu.sync_copy(hrow_v, gh_h.at[pl.ds(NSEL * base, NSEL * bpw)])
        pltpu.sync_copy(trow_v, gt_h.at[pl.ds(NSEL * base, NSEL * bpw)])

    return sc_gather


def _prep_body(l1e_ref, l2e_ref, l1r_ref, l2r_ref, cete_ref, cetr_ref):
    for s in range(S1):
        cete_ref[:, s * S2 * D:(s + 1) * S2 * D] = jnp.dot(
            l1e_ref[s], l2e_ref[...], preferred_element_type=jnp.float32)
        cetr_ref[:, s * S2 * D:(s + 1) * S2 * D] = jnp.dot(
            l1r_ref[s], l2r_ref[...], preferred_element_type=jnp.float32)


def _main_body(time_ref, es_ref, ep_ref, eo_ref, gh_ref, gt_ref,
               cete_ref, cetr_ref,
               s_out, p_out, o_out, h_out, t_out):
    cete = cete_ref[...]
    h_out[...] = jnp.dot(gh_ref[...], cete, preferred_element_type=jnp.float32)
    t_out[...] = jnp.dot(gt_ref[...], cete, preferred_element_type=jnp.float32)
    tm = time_ref[...]
    w = S2 * D

    def timesel(full):
        acc = tm[:, 0:1] * full[:, 0:w]
        for s in range(1, S1):
            acc = acc + tm[:, s:s + 1] * full[:, s * w:(s + 1) * w]
        return acc

    s_out[...] = timesel(
        jnp.dot(es_ref[...], cete, preferred_element_type=jnp.float32))
    o_out[...] = timesel(
        jnp.dot(eo_ref[...], cete, preferred_element_type=jnp.float32))
    p_out[...] = timesel(
        jnp.dot(ep_ref[...], cetr_ref[...], preferred_element_type=jnp.float32))


def kernel(spo, time, nh, nt, entity_embedding, relation_embedding,
           e_layer1, e_layer2, r_layer1, r_layer2):
    B = spo.shape[0]
    ne = entity_embedding.shape[0]
    nr = relation_embedding.shape[0]
    w = S2 * D

    s_idx = spo[:, 0].astype(jnp.int32)
    p_idx = spo[:, 1].astype(jnp.int32)
    o_idx = spo[:, 2].astype(jnp.int32)
    nh32 = nh.astype(jnp.int32)
    nt32 = nt.astype(jnp.int32)
    time32 = time.astype(jnp.float32)

    es, ep, eo, gh, gt = _build_sc_gather(B, ne, nr)(
        s_idx, p_idx, o_idx, time32, nh32, nt32,
        entity_embedding, relation_embedding)

    def blocks_t(l1):
        return jnp.transpose(l1.reshape(S1, D, D), (0, 2, 1))

    cete, cetr = pl.pallas_call(
        _prep_body,
        out_shape=[
            jax.ShapeDtypeStruct((D, S1 * w), jnp.float32),
            jax.ShapeDtypeStruct((D, S1 * w), jnp.float32),
        ],
    )(blocks_t(e_layer1), e_layer2.T, blocks_t(r_layer1), r_layer2.T)

    nblk = B // BLK
    s_o, p_o, o_o, h_o, t_o = pl.pallas_call(
        _main_body,
        grid=(nblk,),
        in_specs=[
            pl.BlockSpec((BLK, 16), lambda i: (i, 0)),
            pl.BlockSpec((BLK, D), lambda i: (i, 0)),
            pl.BlockSpec((BLK, D), lambda i: (i, 0)),
            pl.BlockSpec((BLK, D), lambda i: (i, 0)),
            pl.BlockSpec((NSEL * BLK, D), lambda i: (i, 0)),
            pl.BlockSpec((NSEL * BLK, D), lambda i: (i, 0)),
            pl.BlockSpec((D, S1 * w), lambda i: (0, 0)),
            pl.BlockSpec((D, S1 * w), lambda i: (0, 0)),
        ],
        out_specs=[
            pl.BlockSpec((BLK, w), lambda i: (i, 0)),
            pl.BlockSpec((BLK, w), lambda i: (i, 0)),
            pl.BlockSpec((BLK, w), lambda i: (i, 0)),
            pl.BlockSpec((NSEL * BLK, S1 * w), lambda i: (i, 0)),
            pl.BlockSpec((NSEL * BLK, S1 * w), lambda i: (i, 0)),
        ],
        out_shape=[
            jax.ShapeDtypeStruct((B, w), jnp.float32),
            jax.ShapeDtypeStruct((B, w), jnp.float32),
            jax.ShapeDtypeStruct((B, w), jnp.float32),
            jax.ShapeDtypeStruct((NSEL * B, S1 * w), jnp.float32),
            jax.ShapeDtypeStruct((NSEL * B, S1 * w), jnp.float32),
        ],
    )(time32, es, ep, eo, gh, gt, cete, cetr)

    return (s_o.reshape(B, S2, 1, D),
            p_o.reshape(B, S2, 1, D),
            o_o.reshape(B, S2, 1, D),
            h_o.reshape(B, S2, 4 * NSEL, D),
            t_o.reshape(B, S2, 4 * NSEL, D))
